# baseline (speedup 1.0000x reference)
import numpy as np
import jax
import jax.numpy as jnp
from jax.experimental import pallas as pl
from jax.experimental.pallas import tpu as pltpu

_BINS = 10
_N = 4194304
_TOT = float(_N)
_ROWS = 32768
_COLS = 128
_BLOCK_ROWS = 16384
_CHUNK = 8

_BORDERS = np.arange(_BINS + 1, dtype=np.float32) / _BINS


_GROUP = 32

def _ghm_kernel(p_ref, t_ref, out_ref, acc_ref, mbuf_ref, sbuf_ref):
    pi = pl.program_id(0)
    nprog = pl.num_programs(0)

    zero = jnp.zeros((_CHUNK, _COLS), jnp.float32)
    s_acc = [zero] * _BINS
    c_acc = [zero] * _BINS
    row_idx = jax.lax.broadcasted_iota(
        jnp.int32, (_CHUNK, _GROUP * _CHUNK), 0
    )
    ones_row = jnp.where(row_idx == 0, 1.0, 0.0).astype(jnp.float32)

    for grp in range(_BLOCK_ROWS // (_GROUP * _CHUNK)):
        par = grp % 2
        for c in range(_GROUP):
            j = grp * _GROUP + c
            p = p_ref[pl.ds(j * _CHUNK, _CHUNK), :]
            t = t_ref[pl.ds(j * _CHUNK, _CHUNK), :]
            ap = jnp.abs(p)
            e = jnp.exp(-ap)
            r = 1.0 / (1.0 + e)
            s = jnp.where(p >= 0.0, r, e * r)
            g = jnp.abs(s - t)
            le = jnp.maximum(p, 0.0) - p * t - jnp.log(r)
            s_acc[0] = s_acc[0] + le
            for i in range(1, _BINS):
                m = g >= _BORDERS[i]
                mbuf_ref[par, i - 1, pl.ds(c * _CHUNK, _CHUNK), :] = jnp.where(
                    m, 1.0, 0.0
                )
                sbuf_ref[par, i - 1, pl.ds(c * _CHUNK, _CHUNK), :] = jnp.where(
                    m, le, 0.0
                )
        for i in range(1, _BINS):
            cpart = jax.lax.dot_general(
                ones_row,
                mbuf_ref[par, i - 1],
                (((1,), (0,)), ((), ())),
                preferred_element_type=jnp.float32,
            )
            c_acc[i] = c_acc[i] + cpart
            spart = jax.lax.dot_general(
                ones_row,
                sbuf_ref[par, i - 1],
                (((1,), (0,)), ((), ())),
                preferred_element_type=jnp.float32,
            )
            s_acc[i] = s_acc[i] + spart

    @pl.when(pi == 0)
    def _init():
        for i in range(_BINS):
            acc_ref[i] = s_acc[i]
        for i in range(1, _BINS):
            acc_ref[_BINS + i] = c_acc[i]

    @pl.when(pi > 0)
    def _accum():
        for i in range(_BINS):
            acc_ref[i] += s_acc[i]
        for i in range(1, _BINS):
            acc_ref[_BINS + i] += c_acc[i]

    @pl.when(pi == nprog - 1)
    def _finalize():
        dm = jnp.float32(1.0 - 0.9)
        s_tot = [jnp.sum(acc_ref[i]) for i in range(_BINS)]
        c_tot = [jnp.float32(_TOT)] + [
            jnp.sum(acc_ref[_BINS + i]) for i in range(1, _BINS)
        ]
        loss_acc = jnp.float32(0.0)
        n_count = jnp.float32(0.0)
        for i in range(_BINS):
            c_hi = jnp.float32(0.0) if i == _BINS - 1 else c_tot[i + 1]
            s_hi = jnp.float32(0.0) if i == _BINS - 1 else s_tot[i + 1]
            num = c_tot[i] - c_hi
            sb = s_tot[i] - s_hi
            accm = dm * num
            w = jnp.where(num > 0, _TOT / jnp.maximum(accm, 1e-12), 0.0)
            loss_acc = loss_acc + w * sb
            n_count = n_count + (num > 0).astype(jnp.float32)
        out_ref[0, 0] = loss_acc / jnp.maximum(n_count, 1.0) / _TOT


@jax.jit
def _ghm(pred, target):
    p = pred.reshape(_ROWS, _COLS)
    t = target.astype(jnp.float32).reshape(_ROWS, _COLS)
    grid = (_ROWS // _BLOCK_ROWS,)
    out = pl.pallas_call(
        _ghm_kernel,
        grid=grid,
        in_specs=[
            pl.BlockSpec((_BLOCK_ROWS, _COLS), lambda i: (i, 0)),
            pl.BlockSpec((_BLOCK_ROWS, _COLS), lambda i: (i, 0)),
        ],
        out_specs=pl.BlockSpec(
            (1, 1), lambda i: (0, 0), memory_space=pltpu.SMEM
        ),
        out_shape=jax.ShapeDtypeStruct((1, 1), jnp.float32),
        scratch_shapes=[
            pltpu.VMEM((2 * _BINS, _CHUNK, _COLS), jnp.float32),
            pltpu.VMEM((2, _BINS - 1, _GROUP * _CHUNK, _COLS), jnp.float32),
            pltpu.VMEM((2, _BINS - 1, _GROUP * _CHUNK, _COLS), jnp.float32),
        ],
        compiler_params=pltpu.CompilerParams(
            dimension_semantics=("arbitrary",)
        ),
    )(p, t)
    return out[0, 0]


def kernel(pred, target):
    return _ghm(pred, target)


# speedup vs baseline: 1.0688x; 1.0688x over previous
import numpy as np
import jax
import jax.numpy as jnp
from jax.experimental import pallas as pl
from jax.experimental.pallas import tpu as pltpu

_BINS = 10
_N = 4194304
_TOT = float(_N)
_ROWS = 32768
_COLS = 128
_BLOCK_ROWS = 8192
_CHUNK = 8

_BORDERS = np.arange(_BINS + 1, dtype=np.float32) / _BINS


_GROUP = 32

def _ghm_kernel(p_ref, t_ref, out_ref, acc_ref, mbuf_ref, sbuf_ref):
    pi = pl.program_id(0)
    nprog = pl.num_programs(0)

    zero = jnp.zeros((_CHUNK, _COLS), jnp.float32)
    s_acc = [zero] * _BINS
    c_acc = [zero] * _BINS
    row_idx = jax.lax.broadcasted_iota(
        jnp.int32, (_CHUNK, _GROUP * _CHUNK), 0
    )
    ones_row = jnp.where(row_idx == 0, 1.0, 0.0).astype(jnp.float32)

    for grp in range(_BLOCK_ROWS // (_GROUP * _CHUNK)):
        par = grp % 2
        for c in range(_GROUP):
            j = grp * _GROUP + c
            p = p_ref[pl.ds(j * _CHUNK, _CHUNK), :]
            t = t_ref[pl.ds(j * _CHUNK, _CHUNK), :]
            ap = jnp.abs(p)
            e = jnp.exp(-ap)
            r = 1.0 / (1.0 + e)
            s = jnp.where(p >= 0.0, r, e * r)
            g = jnp.abs(s - t)
            le = jnp.maximum(p, 0.0) - p * t - jnp.log(r)
            s_acc[0] = s_acc[0] + le
            for i in range(1, _BINS):
                m = g >= _BORDERS[i]
                mbuf_ref[par, i - 1, pl.ds(c * _CHUNK, _CHUNK), :] = jnp.where(
                    m, 1.0, 0.0
                )
                sbuf_ref[par, i - 1, pl.ds(c * _CHUNK, _CHUNK), :] = jnp.where(
                    m, le, 0.0
                )
        for i in range(1, _BINS):
            cpart = jax.lax.dot_general(
                ones_row,
                mbuf_ref[par, i - 1],
                (((1,), (0,)), ((), ())),
                preferred_element_type=jnp.float32,
            )
            c_acc[i] = c_acc[i] + cpart
            spart = jax.lax.dot_general(
                ones_row,
                sbuf_ref[par, i - 1],
                (((1,), (0,)), ((), ())),
                preferred_element_type=jnp.float32,
            )
            s_acc[i] = s_acc[i] + spart

    @pl.when(pi == 0)
    def _init():
        for i in range(_BINS):
            acc_ref[i] = s_acc[i]
        for i in range(1, _BINS):
            acc_ref[_BINS + i] = c_acc[i]

    @pl.when(pi > 0)
    def _accum():
        for i in range(_BINS):
            acc_ref[i] += s_acc[i]
        for i in range(1, _BINS):
            acc_ref[_BINS + i] += c_acc[i]

    @pl.when(pi == nprog - 1)
    def _finalize():
        dm = jnp.float32(1.0 - 0.9)
        s_tot = [jnp.sum(acc_ref[i]) for i in range(_BINS)]
        c_tot = [jnp.float32(_TOT)] + [
            jnp.sum(acc_ref[_BINS + i]) for i in range(1, _BINS)
        ]
        loss_acc = jnp.float32(0.0)
        n_count = jnp.float32(0.0)
        for i in range(_BINS):
            c_hi = jnp.float32(0.0) if i == _BINS - 1 else c_tot[i + 1]
            s_hi = jnp.float32(0.0) if i == _BINS - 1 else s_tot[i + 1]
            num = c_tot[i] - c_hi
            sb = s_tot[i] - s_hi
            accm = dm * num
            w = jnp.where(num > 0, _TOT / jnp.maximum(accm, 1e-12), 0.0)
            loss_acc = loss_acc + w * sb
            n_count = n_count + (num > 0).astype(jnp.float32)
        out_ref[0, 0] = loss_acc / jnp.maximum(n_count, 1.0) / _TOT


@jax.jit
def _ghm(pred, target):
    p = pred.reshape(_ROWS, _COLS)
    t = target.astype(jnp.float32).reshape(_ROWS, _COLS)
    grid = (_ROWS // _BLOCK_ROWS,)
    out = pl.pallas_call(
        _ghm_kernel,
        grid=grid,
        in_specs=[
            pl.BlockSpec((_BLOCK_ROWS, _COLS), lambda i: (i, 0)),
            pl.BlockSpec((_BLOCK_ROWS, _COLS), lambda i: (i, 0)),
        ],
        out_specs=pl.BlockSpec(
            (1, 1), lambda i: (0, 0), memory_space=pltpu.SMEM
        ),
        out_shape=jax.ShapeDtypeStruct((1, 1), jnp.float32),
        scratch_shapes=[
            pltpu.VMEM((2 * _BINS, _CHUNK, _COLS), jnp.float32),
            pltpu.VMEM((2, _BINS - 1, _GROUP * _CHUNK, _COLS), jnp.float32),
            pltpu.VMEM((2, _BINS - 1, _GROUP * _CHUNK, _COLS), jnp.float32),
        ],
        compiler_params=pltpu.CompilerParams(
            dimension_semantics=("arbitrary",)
        ),
    )(p, t)
    return out[0, 0]


def kernel(pred, target):
    return _ghm(pred, target)
